# Initial kernel scaffold; baseline (speedup 1.0000x reference)
import jax
import jax.numpy as jnp
from jax.experimental import pallas as pl


def kernel(x, edge_index, edge_attr, u, batch, W1, b1, W2, b2, W3, b3, W4, b4):
    raise NotImplementedError("write your pallas kernel here")


# speedup vs baseline: 2.0975x; 2.0975x over previous
import functools

import jax
import jax.numpy as jnp
from jax import lax
from jax.experimental import pallas as pl
from jax.experimental.pallas import tpu as pltpu
from jax.experimental.pallas import tpu_sc as plsc

N_NODES = 10000
N_EDGES = 320000
D = 128
D_EDGE = 16
D_GLOBAL = 16
N_GRAPHS = 8
DC = D + 16
NC, NS = 2, 16
NW = NC * NS
EPW = N_EDGES // NW
CH = 80
NCHUNK = EPW // CH
N_PAD = 10240
ROWS_PER_SUB = N_PAD // NS


def _mm_kernel(a_ref, b_ref, o_ref):
    o_ref[...] = jnp.dot(a_ref[...], b_ref[...], preferred_element_type=jnp.float32)


def _edge_pre_kernel(ea_ref, w_ref, b_ref, o_ref):
    o_ref[...] = (
        jnp.dot(ea_ref[...], w_ref[...], preferred_element_type=jnp.float32)
        + b_ref[...]
    )


def _sc_body(xw1_hbm, eb_hbm, row_hbm, col_hbm, out_hbm,
             acc_sh, row_v, col_v, eb_v, g_v, h_v, sem):
    cid = lax.axis_index("c")
    sid = lax.axis_index("s")
    wid = sid * NC + cid

    zero16 = jnp.zeros((16,), jnp.float32)
    one16 = jnp.ones((16,), jnp.float32)

    def zfill(i, _):
        for j in range(DC // 16):
            h_v[i, pl.ds(j * 16, 16)] = zero16
        return 0
    lax.fori_loop(0, CH, zfill, 0)

    def zcopy(i, _):
        pltpu.sync_copy(h_v, acc_sh.at[pl.ds(sid * ROWS_PER_SUB + i * CH, CH), :])
        return 0
    lax.fori_loop(0, ROWS_PER_SUB // CH, zcopy, 0)

    def onefill(i, _):
        h_v[i, pl.ds(D, 16)] = one16
        return 0
    lax.fori_loop(0, CH, onefill, 0)
    plsc.subcore_barrier()

    def chunk(k, _):
        base = wid * EPW + k * CH
        pltpu.sync_copy(row_hbm.at[pl.ds(base, CH)], row_v)
        pltpu.sync_copy(col_hbm.at[pl.ds(base, CH)], col_v)
        pltpu.async_copy(xw1_hbm.at[row_v], g_v, sem).wait()
        pltpu.sync_copy(eb_hbm.at[pl.ds(base, CH), :], eb_v)

        def fuse(i, _):
            for j in range(D // 16):
                sl = pl.ds(j * 16, 16)
                h_v[i, sl] = jnp.maximum(g_v[i, sl] + eb_v[i, sl], 0.0)
            return 0
        lax.fori_loop(0, CH, fuse, 0)

        pltpu.sync_copy(h_v, acc_sh.at[col_v], add=True)
        return 0
    lax.fori_loop(0, NCHUNK, chunk, 0)

    plsc.subcore_barrier()
    pltpu.sync_copy(
        acc_sh.at[pl.ds(sid * ROWS_PER_SUB, ROWS_PER_SUB), :],
        out_hbm.at[cid, pl.ds(sid * ROWS_PER_SUB, ROWS_PER_SUB), :],
    )


def _node_mlp_kernel(x_ref, s_ref, batch_ref, u_ref, w2_ref, b2_ref,
                     w3_ref, b3_ref, w4_ref, b4_ref, o_ref):
    s = s_ref[0, :, :D] + s_ref[1, :, :D]
    c = (s_ref[0, :, D:D + 1] + s_ref[1, :, D:D + 1])
    inv = 1.0 / jnp.maximum(c, 1.0)
    mean = (jnp.dot(s, w2_ref[...], preferred_element_type=jnp.float32)
            + c * b2_ref[...]) * inv
    b = batch_ref[0, 0, :]
    oh = (b[:, None] == lax.broadcasted_iota(jnp.int32, (b.shape[0], N_GRAPHS), 1))
    uc = jnp.dot(u_ref[...], w3_ref[D + D:, :], preferred_element_type=jnp.float32)
    t = (jnp.dot(x_ref[...], w3_ref[:D, :], preferred_element_type=jnp.float32)
         + jnp.dot(mean, w3_ref[D:D + D, :], preferred_element_type=jnp.float32)
         + jnp.dot(oh.astype(jnp.float32), uc, preferred_element_type=jnp.float32)
         + b3_ref[...])
    o_ref[...] = (jnp.dot(jnp.maximum(t, 0.0), w4_ref[...],
                          preferred_element_type=jnp.float32) + b4_ref[...])


def kernel(x, edge_index, edge_attr, u, batch, W1, b1, W2, b2, W3, b3, W4, b4):
    row = edge_index[0].astype(jnp.int32)
    col = edge_index[1].astype(jnp.int32)
    W1a = W1[:D]
    W1b = W1[D:]

    xw1 = pl.pallas_call(
        _mm_kernel,
        grid=(5,),
        in_specs=[
            pl.BlockSpec((N_NODES // 5, D), lambda i: (i, 0)),
            pl.BlockSpec((D, D), lambda i: (0, 0)),
        ],
        out_specs=pl.BlockSpec((N_NODES // 5, D), lambda i: (i, 0)),
        out_shape=jax.ShapeDtypeStruct((N_NODES, D), jnp.float32),
    )(x, W1a)

    EB_BLK = 4000
    eb = pl.pallas_call(
        _edge_pre_kernel,
        grid=(N_EDGES // EB_BLK,),
        in_specs=[
            pl.BlockSpec((EB_BLK, D_EDGE), lambda i: (i, 0)),
            pl.BlockSpec((D_EDGE, D), lambda i: (0, 0)),
            pl.BlockSpec((D,), lambda i: (0,)),
        ],
        out_specs=pl.BlockSpec((EB_BLK, D), lambda i: (i, 0)),
        out_shape=jax.ShapeDtypeStruct((N_EDGES, D), jnp.float32),
    )(edge_attr, W1b, b1)

    mesh = plsc.VectorSubcoreMesh(core_axis_name="c", subcore_axis_name="s")
    sc = functools.partial(
        pl.kernel,
        mesh=mesh,
        out_type=jax.ShapeDtypeStruct((NC, N_PAD, DC), jnp.float32),
        scratch_types=[
            pltpu.VMEM_SHARED((N_PAD, DC), jnp.float32),
            pltpu.VMEM((CH,), jnp.int32),
            pltpu.VMEM((CH,), jnp.int32),
            pltpu.VMEM((CH, D), jnp.float32),
            pltpu.VMEM((CH, D), jnp.float32),
            pltpu.VMEM((CH, DC), jnp.float32),
            pltpu.SemaphoreType.DMA,
        ],
        compiler_params=pltpu.CompilerParams(use_tc_tiling_on_sc=False),
    )(_sc_body)
    s01 = sc(xw1, eb, row, col)

    R = 1000
    batch3 = batch.astype(jnp.int32).reshape(N_NODES // R, 1, R)
    out = pl.pallas_call(
        _node_mlp_kernel,
        grid=(N_NODES // R,),
        in_specs=[
            pl.BlockSpec((R, D), lambda i: (i, 0)),
            pl.BlockSpec((NC, R, DC), lambda i: (0, i, 0)),
            pl.BlockSpec((1, 1, R), lambda i: (i, 0, 0)),
            pl.BlockSpec((N_GRAPHS, D_GLOBAL), lambda i: (0, 0)),
            pl.BlockSpec((D, D), lambda i: (0, 0)),
            pl.BlockSpec((D,), lambda i: (0,)),
            pl.BlockSpec((D + D + D_GLOBAL, D), lambda i: (0, 0)),
            pl.BlockSpec((D,), lambda i: (0,)),
            pl.BlockSpec((D, D), lambda i: (0, 0)),
            pl.BlockSpec((D,), lambda i: (0,)),
        ],
        out_specs=pl.BlockSpec((R, D), lambda i: (i, 0)),
        out_shape=jax.ShapeDtypeStruct((N_NODES, D), jnp.float32),
    )(x, s01, batch3, u, W2, b2, W3, b3, W4, b4)
    return out


# speedup vs baseline: 2.1333x; 1.0171x over previous
import functools

import jax
import jax.numpy as jnp
from jax import lax
from jax.experimental import pallas as pl
from jax.experimental.pallas import tpu as pltpu
from jax.experimental.pallas import tpu_sc as plsc

N_NODES = 10000
N_EDGES = 320000
D = 128
D_EDGE = 16
D_GLOBAL = 16
N_GRAPHS = 8
DC = D + 16
NC, NS = 2, 16
NW = NC * NS
EPW = N_EDGES // NW
CH = 40
NCHUNK = EPW // CH
NB = 2
CHB = 50
NBATCH = NCHUNK // CHB
N_PAD = 10240
ROWS_PER_SUB = N_PAD // NS


def _mm_kernel(a_ref, b_ref, o_ref):
    o_ref[...] = jnp.dot(a_ref[...], b_ref[...], preferred_element_type=jnp.float32)


def _edge_pre_kernel(ea_ref, w_ref, b_ref, o_ref):
    m = (jnp.dot(ea_ref[...], w_ref[...], preferred_element_type=jnp.float32)
         + b_ref[...])
    o_ref[...] = jnp.concatenate(
        [m, jnp.ones((m.shape[0], DC - D), jnp.float32)], axis=1)


def _sc_body(xw1_hbm, eb_hbm, row_hbm, col_hbm, out_hbm,
             acc_sh, row_b, col_b, g0, g1, h0, h1, gs0, gs1, es0, es1,
             ss0, ss1):
    cid = lax.axis_index("c")
    sid = lax.axis_index("s")
    wid = sid * NC + cid
    g = (g0, g1)
    h = (h0, h1)
    gsem = (gs0, gs1)
    esem = (es0, es1)
    ssem = (ss0, ss1)

    zero16 = jnp.zeros((16,), jnp.float32)

    def zfill(i, _):
        for j in range(DC // 16):
            h0[i, pl.ds(j * 16, 16)] = zero16
        return 0
    lax.fori_loop(0, CH, zfill, 0)

    def zcopy(i, _):
        pltpu.sync_copy(h0, acc_sh.at[pl.ds(sid * ROWS_PER_SUB + i * CH, CH), :])
        return 0
    lax.fori_loop(0, ROWS_PER_SUB // CH, zcopy, 0)
    plsc.subcore_barrier()

    def issue(c, b):
        ebase = _ebase(c)
        pltpu.async_copy(eb_hbm.at[pl.ds(ebase, CH), :], h[b], esem[b])
        pltpu.async_copy(xw1_hbm.at[row_b.at[c]], g[b], gsem[b])

    for bt in range(NBATCH):
        cbase = wid * NCHUNK + bt * CHB

        def _ebase(c):
            return (cbase + c) * CH

        pltpu.sync_copy(row_hbm.at[pl.ds(cbase, CHB), :], row_b)
        pltpu.sync_copy(col_hbm.at[pl.ds(cbase, CHB), :], col_b)
        for b in range(NB):
            issue(b, b)

        def grp(i, _):
            for b in range(NB):
                c = i * NB + b
                pltpu.make_async_copy(eb_hbm.at[pl.ds(0, CH), :], h[b],
                                      esem[b]).wait()
                pltpu.make_async_copy(xw1_hbm.at[row_b.at[c]], g[b],
                                      gsem[b]).wait()

                def fuse(r, _):
                    for j in range(D // 16):
                        sl = pl.ds(j * 16, 16)
                        h[b][r, sl] = jnp.maximum(h[b][r, sl] + g[b][r, sl],
                                                  0.0)
                    return 0
                lax.fori_loop(0, CH, fuse, 0)

                pltpu.async_copy(h[b], acc_sh.at[col_b.at[c]], ssem[b],
                                 add=True).wait()

                @pl.when(c + NB < CHB)
                def _():
                    issue(c + NB, b)
            return 0
        lax.fori_loop(0, CHB // NB, grp, 0)

    plsc.subcore_barrier()
    pltpu.sync_copy(
        acc_sh.at[pl.ds(sid * ROWS_PER_SUB, ROWS_PER_SUB), :],
        out_hbm.at[cid, pl.ds(sid * ROWS_PER_SUB, ROWS_PER_SUB), :],
    )


def _node_mlp_kernel(x_ref, s_ref, batch_ref, u_ref, w2_ref, b2_ref,
                     w3_ref, b3_ref, w4_ref, b4_ref, o_ref):
    s = s_ref[0, :, :D] + s_ref[1, :, :D]
    c = (s_ref[0, :, D:D + 1] + s_ref[1, :, D:D + 1])
    inv = 1.0 / jnp.maximum(c, 1.0)
    mean = (jnp.dot(s, w2_ref[...], preferred_element_type=jnp.float32)
            + c * b2_ref[...]) * inv
    b = batch_ref[0, 0, :]
    oh = (b[:, None] == lax.broadcasted_iota(jnp.int32, (b.shape[0], N_GRAPHS), 1))
    uc = jnp.dot(u_ref[...], w3_ref[D + D:, :], preferred_element_type=jnp.float32)
    t = (jnp.dot(x_ref[...], w3_ref[:D, :], preferred_element_type=jnp.float32)
         + jnp.dot(mean, w3_ref[D:D + D, :], preferred_element_type=jnp.float32)
         + jnp.dot(oh.astype(jnp.float32), uc, preferred_element_type=jnp.float32)
         + b3_ref[...])
    o_ref[...] = (jnp.dot(jnp.maximum(t, 0.0), w4_ref[...],
                          preferred_element_type=jnp.float32) + b4_ref[...])


def kernel(x, edge_index, edge_attr, u, batch, W1, b1, W2, b2, W3, b3, W4, b4):
    row = edge_index[0].astype(jnp.int32)
    col = edge_index[1].astype(jnp.int32)
    W1a = W1[:D]
    W1b = W1[D:]

    xw1 = pl.pallas_call(
        _mm_kernel,
        grid=(5,),
        in_specs=[
            pl.BlockSpec((N_NODES // 5, D), lambda i: (i, 0)),
            pl.BlockSpec((D, D), lambda i: (0, 0)),
        ],
        out_specs=pl.BlockSpec((N_NODES // 5, D), lambda i: (i, 0)),
        out_shape=jax.ShapeDtypeStruct((N_NODES, D), jnp.float32),
    )(x, W1a)

    EB_BLK = 4000
    eb = pl.pallas_call(
        _edge_pre_kernel,
        grid=(N_EDGES // EB_BLK,),
        in_specs=[
            pl.BlockSpec((EB_BLK, D_EDGE), lambda i: (i, 0)),
            pl.BlockSpec((D_EDGE, D), lambda i: (0, 0)),
            pl.BlockSpec((D,), lambda i: (0,)),
        ],
        out_specs=pl.BlockSpec((EB_BLK, DC), lambda i: (i, 0)),
        out_shape=jax.ShapeDtypeStruct((N_EDGES, DC), jnp.float32),
    )(edge_attr, W1b, b1)

    mesh = plsc.VectorSubcoreMesh(core_axis_name="c", subcore_axis_name="s")
    sc = functools.partial(
        pl.kernel,
        mesh=mesh,
        out_type=jax.ShapeDtypeStruct((NC, N_PAD, DC), jnp.float32),
        scratch_types=[
            pltpu.VMEM_SHARED((N_PAD, DC), jnp.float32),
            pltpu.VMEM((CHB, CH), jnp.int32),
            pltpu.VMEM((CHB, CH), jnp.int32),
            pltpu.VMEM((CH, D), jnp.float32),
            pltpu.VMEM((CH, D), jnp.float32),
            pltpu.VMEM((CH, DC), jnp.float32),
            pltpu.VMEM((CH, DC), jnp.float32),
            pltpu.SemaphoreType.DMA,
            pltpu.SemaphoreType.DMA,
            pltpu.SemaphoreType.DMA,
            pltpu.SemaphoreType.DMA,
            pltpu.SemaphoreType.DMA,
            pltpu.SemaphoreType.DMA,
        ],
        compiler_params=pltpu.CompilerParams(use_tc_tiling_on_sc=False),
    )(_sc_body)
    s01 = sc(xw1, eb, row.reshape(N_EDGES // CH, CH), col.reshape(N_EDGES // CH, CH))

    R = 1000
    batch3 = batch.astype(jnp.int32).reshape(N_NODES // R, 1, R)
    out = pl.pallas_call(
        _node_mlp_kernel,
        grid=(N_NODES // R,),
        in_specs=[
            pl.BlockSpec((R, D), lambda i: (i, 0)),
            pl.BlockSpec((NC, R, DC), lambda i: (0, i, 0)),
            pl.BlockSpec((1, 1, R), lambda i: (i, 0, 0)),
            pl.BlockSpec((N_GRAPHS, D_GLOBAL), lambda i: (0, 0)),
            pl.BlockSpec((D, D), lambda i: (0, 0)),
            pl.BlockSpec((D,), lambda i: (0,)),
            pl.BlockSpec((D + D + D_GLOBAL, D), lambda i: (0, 0)),
            pl.BlockSpec((D,), lambda i: (0,)),
            pl.BlockSpec((D, D), lambda i: (0, 0)),
            pl.BlockSpec((D,), lambda i: (0,)),
        ],
        out_specs=pl.BlockSpec((R, D), lambda i: (i, 0)),
        out_shape=jax.ShapeDtypeStruct((N_NODES, D), jnp.float32),
    )(x, s01, batch3, u, W2, b2, W3, b3, W4, b4)
    return out


# speedup vs baseline: 2.6560x; 1.2450x over previous
import functools

import jax
import jax.numpy as jnp
from jax import lax
from jax.experimental import pallas as pl
from jax.experimental.pallas import tpu as pltpu
from jax.experimental.pallas import tpu_sc as plsc

N_NODES = 10000
N_EDGES = 320000
D = 128
D_EDGE = 16
D_GLOBAL = 16
N_GRAPHS = 8
DC = D + 16
NC, NS = 2, 16
NW = NC * NS
EPW = N_EDGES // NW
CH = 128
NROW = N_EDGES // CH
N_PAD = 10240
ROWS_PER_SUB = N_PAD // NS


def _mm_kernel(a_ref, b_ref, o_ref):
    o_ref[...] = jnp.dot(a_ref[...], b_ref[...], preferred_element_type=jnp.float32)


def _edge_pre_kernel(ea_ref, w_ref, b_ref, o_ref):
    o_ref[...] = (
        jnp.dot(ea_ref[...], w_ref[...], preferred_element_type=jnp.float32)
        + b_ref[...])


def _sc_body(xw1_hbm, eb_hbm, row_hbm, col_hbm, out_hbm,
             acc_sh, row_v, col_v, g_v, h_v, rs0, rs1, gsem, esem, ssem):
    cid = lax.axis_index("c")
    sid = lax.axis_index("s")
    wid = sid * NC + cid
    rsem = (rs0, rs1)

    zero16 = jnp.zeros((16,), jnp.float32)
    one16 = jnp.ones((16,), jnp.float32)

    def zfill(i, _):
        for j in range(DC // 16):
            h_v[i, pl.ds(j * 16, 16)] = zero16
        return 0
    lax.fori_loop(0, CH, zfill, 0)

    def zcopy(i, _):
        pltpu.sync_copy(h_v, acc_sh.at[pl.ds(sid * ROWS_PER_SUB + i * CH, CH), :])
        return 0
    lax.fori_loop(0, ROWS_PER_SUB // CH, zcopy, 0)

    def onefill(i, _):
        h_v[i, pl.ds(D, 16)] = one16
        return 0
    lax.fori_loop(0, CH, onefill, 0)
    plsc.subcore_barrier()

    nrows = jnp.where(wid < 4, NROW // NW + 1, NROW // NW)
    rstart = NROW // NW * wid + jnp.minimum(wid, 4)

    def idx_issue(r, b):
        pltpu.async_copy(row_hbm.at[pl.ds(r, 1), :], row_v.at[pl.ds(b, 1), :],
                         rsem[b])
        pltpu.async_copy(col_hbm.at[pl.ds(r, 1), :], col_v.at[pl.ds(b, 1), :],
                         rsem[b])

    idx_issue(rstart, 0)

    def chunk(c, _):
        r = rstart + c
        b = lax.rem(c, 2)
        pltpu.async_copy(eb_hbm.at[pl.ds(r * CH, CH), :],
                         h_v.at[:, pl.ds(0, D)], esem)
        for bb in range(2):
            @pl.when(b == bb)
            def _():
                pltpu.make_async_copy(row_hbm.at[pl.ds(0, 1), :],
                                      row_v.at[pl.ds(bb, 1), :], rsem[bb]).wait()
                pltpu.make_async_copy(col_hbm.at[pl.ds(0, 1), :],
                                      col_v.at[pl.ds(bb, 1), :], rsem[bb]).wait()
                pltpu.async_copy(xw1_hbm.at[row_v.at[bb]], g_v, gsem)

        @pl.when(c + 1 < nrows)
        def _():
            for bb in range(2):
                @pl.when(b == bb)
                def _():
                    idx_issue(r + 1, 1 - bb)

        pltpu.make_async_copy(eb_hbm.at[pl.ds(0, CH), :],
                              h_v.at[:, pl.ds(0, D)], esem).wait()
        pltpu.make_async_copy(xw1_hbm.at[row_v.at[0]], g_v, gsem).wait()

        def fuse(i, _):
            for j in range(D // 16):
                sl = pl.ds(j * 16, 16)
                h_v[i, sl] = jnp.maximum(h_v[i, sl] + g_v[i, sl], 0.0)
            return 0
        lax.fori_loop(0, CH, fuse, 0)

        for bb in range(2):
            @pl.when(b == bb)
            def _():
                pltpu.async_copy(h_v, acc_sh.at[col_v.at[bb]], ssem,
                                 add=True).wait()
        return 0
    lax.fori_loop(0, nrows, chunk, 0)

    plsc.subcore_barrier()
    pltpu.sync_copy(
        acc_sh.at[pl.ds(sid * ROWS_PER_SUB, ROWS_PER_SUB), :],
        out_hbm.at[cid, pl.ds(sid * ROWS_PER_SUB, ROWS_PER_SUB), :],
    )


def _node_mlp_kernel(x_ref, s_ref, batch_ref, u_ref, w2_ref, b2_ref,
                     w3_ref, b3_ref, w4_ref, b4_ref, o_ref):
    s = s_ref[0, :, :D] + s_ref[1, :, :D]
    c = (s_ref[0, :, D:D + 1] + s_ref[1, :, D:D + 1])
    inv = 1.0 / jnp.maximum(c, 1.0)
    mean = (jnp.dot(s, w2_ref[...], preferred_element_type=jnp.float32)
            + c * b2_ref[...]) * inv
    b = batch_ref[0, 0, :]
    oh = (b[:, None] == lax.broadcasted_iota(jnp.int32, (b.shape[0], N_GRAPHS), 1))
    uc = jnp.dot(u_ref[...], w3_ref[D + D:, :], preferred_element_type=jnp.float32)
    t = (jnp.dot(x_ref[...], w3_ref[:D, :], preferred_element_type=jnp.float32)
         + jnp.dot(mean, w3_ref[D:D + D, :], preferred_element_type=jnp.float32)
         + jnp.dot(oh.astype(jnp.float32), uc, preferred_element_type=jnp.float32)
         + b3_ref[...])
    o_ref[...] = (jnp.dot(jnp.maximum(t, 0.0), w4_ref[...],
                          preferred_element_type=jnp.float32) + b4_ref[...])


def kernel(x, edge_index, edge_attr, u, batch, W1, b1, W2, b2, W3, b3, W4, b4):
    row = edge_index[0].astype(jnp.int32)
    col = edge_index[1].astype(jnp.int32)
    W1a = W1[:D]
    W1b = W1[D:]

    xw1 = pl.pallas_call(
        _mm_kernel,
        grid=(5,),
        in_specs=[
            pl.BlockSpec((N_NODES // 5, D), lambda i: (i, 0)),
            pl.BlockSpec((D, D), lambda i: (0, 0)),
        ],
        out_specs=pl.BlockSpec((N_NODES // 5, D), lambda i: (i, 0)),
        out_shape=jax.ShapeDtypeStruct((N_NODES, D), jnp.float32),
    )(x, W1a)

    EB_BLK = 4000
    eb = pl.pallas_call(
        _edge_pre_kernel,
        grid=(N_EDGES // EB_BLK,),
        in_specs=[
            pl.BlockSpec((EB_BLK, D_EDGE), lambda i: (i, 0)),
            pl.BlockSpec((D_EDGE, D), lambda i: (0, 0)),
            pl.BlockSpec((D,), lambda i: (0,)),
        ],
        out_specs=pl.BlockSpec((EB_BLK, D), lambda i: (i, 0)),
        out_shape=jax.ShapeDtypeStruct((N_EDGES, D), jnp.float32),
    )(edge_attr, W1b, b1)

    mesh = plsc.VectorSubcoreMesh(core_axis_name="c", subcore_axis_name="s")
    sc = functools.partial(
        pl.kernel,
        mesh=mesh,
        out_type=jax.ShapeDtypeStruct((NC, N_PAD, DC), jnp.float32),
        scratch_types=[
            pltpu.VMEM_SHARED((N_PAD, DC), jnp.float32),
            pltpu.VMEM((2, CH), jnp.int32),
            pltpu.VMEM((2, CH), jnp.int32),
            pltpu.VMEM((CH, D), jnp.float32),
            pltpu.VMEM((CH, DC), jnp.float32),
            pltpu.SemaphoreType.DMA,
            pltpu.SemaphoreType.DMA,
            pltpu.SemaphoreType.DMA,
            pltpu.SemaphoreType.DMA,
            pltpu.SemaphoreType.DMA,
        ],
        compiler_params=pltpu.CompilerParams(use_tc_tiling_on_sc=False),
    )(_sc_body)
    s01 = sc(xw1, eb, row.reshape(NROW, CH), col.reshape(NROW, CH))

    R = 1000
    batch3 = batch.astype(jnp.int32).reshape(N_NODES // R, 1, R)
    out = pl.pallas_call(
        _node_mlp_kernel,
        grid=(N_NODES // R,),
        in_specs=[
            pl.BlockSpec((R, D), lambda i: (i, 0)),
            pl.BlockSpec((NC, R, DC), lambda i: (0, i, 0)),
            pl.BlockSpec((1, 1, R), lambda i: (i, 0, 0)),
            pl.BlockSpec((N_GRAPHS, D_GLOBAL), lambda i: (0, 0)),
            pl.BlockSpec((D, D), lambda i: (0, 0)),
            pl.BlockSpec((D,), lambda i: (0,)),
            pl.BlockSpec((D + D + D_GLOBAL, D), lambda i: (0, 0)),
            pl.BlockSpec((D,), lambda i: (0,)),
            pl.BlockSpec((D, D), lambda i: (0, 0)),
            pl.BlockSpec((D,), lambda i: (0,)),
        ],
        out_specs=pl.BlockSpec((R, D), lambda i: (i, 0)),
        out_shape=jax.ShapeDtypeStruct((N_NODES, D), jnp.float32),
    )(x, s01, batch3, u, W2, b2, W3, b3, W4, b4)
    return out


# speedup vs baseline: 4.0817x; 1.5368x over previous
import functools

import jax
import jax.numpy as jnp
from jax import lax
from jax.experimental import pallas as pl
from jax.experimental.pallas import tpu as pltpu
from jax.experimental.pallas import tpu_sc as plsc

N_NODES = 10000
N_EDGES = 320000
D = 128
D_EDGE = 16
D_GLOBAL = 16
N_GRAPHS = 8
DC = D + 16
NC, NS = 2, 16
NW = NC * NS
EPW = N_EDGES // NW
CH = 128
NROW = N_EDGES // CH
N_PAD = 10240
ROWS_PER_SUB = N_PAD // NS


def _mm_kernel(a_ref, b_ref, o_ref):
    o_ref[...] = jnp.dot(a_ref[...], b_ref[...], preferred_element_type=jnp.float32)


def _edge_pre_kernel(ea_ref, w_ref, b_ref, o_ref):
    o_ref[...] = (
        jnp.dot(ea_ref[...], w_ref[...], preferred_element_type=jnp.float32)
        + b_ref[...])


def _sc_body(xw1_hbm, eb_hbm, row_hbm, col_hbm, out_hbm,
             acc_sh, row_v, col_v, g_v, h_v, rs0, rs1, gsem, esem, ssem):
    cid = lax.axis_index("c")
    sid = lax.axis_index("s")
    wid = sid * NC + cid
    rsem = (rs0, rs1)

    zero16 = jnp.zeros((16,), jnp.float32)
    one16 = jnp.ones((16,), jnp.float32)

    def zfill(i, _):
        for j in range(DC // 16):
            h_v[i, pl.ds(j * 16, 16)] = zero16
        return 0
    lax.fori_loop(0, CH, zfill, 0)

    def zcopy(i, _):
        pltpu.sync_copy(h_v, acc_sh.at[pl.ds(sid * ROWS_PER_SUB + i * CH, CH), :])
        return 0
    lax.fori_loop(0, ROWS_PER_SUB // CH, zcopy, 0)

    def onefill(i, _):
        h_v[i, pl.ds(D, 16)] = one16
        return 0
    lax.fori_loop(0, CH, onefill, 0)
    plsc.subcore_barrier()

    nrows = jnp.where(wid < 4, NROW // NW + 1, NROW // NW)
    rstart = NROW // NW * wid + jnp.minimum(wid, 4)

    def idx_issue(r, b):
        pltpu.async_copy(row_hbm.at[pl.ds(r, 1), :], row_v.at[pl.ds(b, 1), :],
                         rsem[b])
        pltpu.async_copy(col_hbm.at[pl.ds(r, 1), :], col_v.at[pl.ds(b, 1), :],
                         rsem[b])

    idx_issue(rstart, 0)

    def chunk(c, _):
        r = rstart + c
        b = lax.rem(c, 2)
        pltpu.async_copy(eb_hbm.at[pl.ds(r * CH, CH), :],
                         h_v.at[:, pl.ds(0, D)], esem)
        for bb in range(2):
            @pl.when(b == bb)
            def _():
                pltpu.make_async_copy(row_hbm.at[pl.ds(0, 1), :],
                                      row_v.at[pl.ds(bb, 1), :], rsem[bb]).wait()
                pltpu.make_async_copy(col_hbm.at[pl.ds(0, 1), :],
                                      col_v.at[pl.ds(bb, 1), :], rsem[bb]).wait()
                pltpu.async_copy(xw1_hbm.at[row_v.at[bb]], g_v, gsem)

        @pl.when(c + 1 < nrows)
        def _():
            for bb in range(2):
                @pl.when(b == bb)
                def _():
                    idx_issue(r + 1, 1 - bb)

        pltpu.make_async_copy(eb_hbm.at[pl.ds(0, CH), :],
                              h_v.at[:, pl.ds(0, D)], esem).wait()
        pltpu.make_async_copy(xw1_hbm.at[row_v.at[0]], g_v, gsem).wait()

        @plsc.parallel_loop(0, CH, 1, unroll=4)
        def fuse(i):
            for j in range(D // 16):
                sl = pl.ds(j * 16, 16)
                h_v[i, sl] = jnp.maximum(h_v[i, sl] + g_v[i, sl], 0.0)

        for bb in range(2):
            @pl.when(b == bb)
            def _():
                pltpu.async_copy(h_v, acc_sh.at[col_v.at[bb]], ssem,
                                 add=True).wait()
        return 0
    lax.fori_loop(0, nrows, chunk, 0)

    plsc.subcore_barrier()
    pltpu.sync_copy(
        acc_sh.at[pl.ds(sid * ROWS_PER_SUB, ROWS_PER_SUB), :],
        out_hbm.at[cid, pl.ds(sid * ROWS_PER_SUB, ROWS_PER_SUB), :],
    )


def _node_mlp_kernel(x_ref, s_ref, batch_ref, u_ref, w2_ref, b2_ref,
                     w3_ref, b3_ref, w4_ref, b4_ref, o_ref):
    s = s_ref[0, :, :D] + s_ref[1, :, :D]
    c = (s_ref[0, :, D:D + 1] + s_ref[1, :, D:D + 1])
    inv = 1.0 / jnp.maximum(c, 1.0)
    mean = (jnp.dot(s, w2_ref[...], preferred_element_type=jnp.float32)
            + c * b2_ref[...]) * inv
    b = batch_ref[0, 0, :]
    oh = (b[:, None] == lax.broadcasted_iota(jnp.int32, (b.shape[0], N_GRAPHS), 1))
    uc = jnp.dot(u_ref[...], w3_ref[D + D:, :], preferred_element_type=jnp.float32)
    t = (jnp.dot(x_ref[...], w3_ref[:D, :], preferred_element_type=jnp.float32)
         + jnp.dot(mean, w3_ref[D:D + D, :], preferred_element_type=jnp.float32)
         + jnp.dot(oh.astype(jnp.float32), uc, preferred_element_type=jnp.float32)
         + b3_ref[...])
    o_ref[...] = (jnp.dot(jnp.maximum(t, 0.0), w4_ref[...],
                          preferred_element_type=jnp.float32) + b4_ref[...])


def kernel(x, edge_index, edge_attr, u, batch, W1, b1, W2, b2, W3, b3, W4, b4):
    row = edge_index[0].astype(jnp.int32)
    col = edge_index[1].astype(jnp.int32)
    W1a = W1[:D]
    W1b = W1[D:]

    xw1 = pl.pallas_call(
        _mm_kernel,
        grid=(5,),
        in_specs=[
            pl.BlockSpec((N_NODES // 5, D), lambda i: (i, 0)),
            pl.BlockSpec((D, D), lambda i: (0, 0)),
        ],
        out_specs=pl.BlockSpec((N_NODES // 5, D), lambda i: (i, 0)),
        out_shape=jax.ShapeDtypeStruct((N_NODES, D), jnp.float32),
    )(x, W1a)

    EB_BLK = 4000
    eb = pl.pallas_call(
        _edge_pre_kernel,
        grid=(N_EDGES // EB_BLK,),
        in_specs=[
            pl.BlockSpec((EB_BLK, D_EDGE), lambda i: (i, 0)),
            pl.BlockSpec((D_EDGE, D), lambda i: (0, 0)),
            pl.BlockSpec((D,), lambda i: (0,)),
        ],
        out_specs=pl.BlockSpec((EB_BLK, D), lambda i: (i, 0)),
        out_shape=jax.ShapeDtypeStruct((N_EDGES, D), jnp.float32),
    )(edge_attr, W1b, b1)

    mesh = plsc.VectorSubcoreMesh(core_axis_name="c", subcore_axis_name="s")
    sc = functools.partial(
        pl.kernel,
        mesh=mesh,
        out_type=jax.ShapeDtypeStruct((NC, N_PAD, DC), jnp.float32),
        scratch_types=[
            pltpu.VMEM_SHARED((N_PAD, DC), jnp.float32),
            pltpu.VMEM((2, CH), jnp.int32),
            pltpu.VMEM((2, CH), jnp.int32),
            pltpu.VMEM((CH, D), jnp.float32),
            pltpu.VMEM((CH, DC), jnp.float32),
            pltpu.SemaphoreType.DMA,
            pltpu.SemaphoreType.DMA,
            pltpu.SemaphoreType.DMA,
            pltpu.SemaphoreType.DMA,
            pltpu.SemaphoreType.DMA,
        ],
        compiler_params=pltpu.CompilerParams(use_tc_tiling_on_sc=False),
    )(_sc_body)
    s01 = sc(xw1, eb, row.reshape(NROW, CH), col.reshape(NROW, CH))

    R = 1000
    batch3 = batch.astype(jnp.int32).reshape(N_NODES // R, 1, R)
    out = pl.pallas_call(
        _node_mlp_kernel,
        grid=(N_NODES // R,),
        in_specs=[
            pl.BlockSpec((R, D), lambda i: (i, 0)),
            pl.BlockSpec((NC, R, DC), lambda i: (0, i, 0)),
            pl.BlockSpec((1, 1, R), lambda i: (i, 0, 0)),
            pl.BlockSpec((N_GRAPHS, D_GLOBAL), lambda i: (0, 0)),
            pl.BlockSpec((D, D), lambda i: (0, 0)),
            pl.BlockSpec((D,), lambda i: (0,)),
            pl.BlockSpec((D + D + D_GLOBAL, D), lambda i: (0, 0)),
            pl.BlockSpec((D,), lambda i: (0,)),
            pl.BlockSpec((D, D), lambda i: (0, 0)),
            pl.BlockSpec((D,), lambda i: (0,)),
        ],
        out_specs=pl.BlockSpec((R, D), lambda i: (i, 0)),
        out_shape=jax.ShapeDtypeStruct((N_NODES, D), jnp.float32),
    )(x, s01, batch3, u, W2, b2, W3, b3, W4, b4)
    return out


# speedup vs baseline: 5.1067x; 1.2511x over previous
import functools

import jax
import jax.numpy as jnp
from jax import lax
from jax.experimental import pallas as pl
from jax.experimental.pallas import tpu as pltpu
from jax.experimental.pallas import tpu_sc as plsc

N_NODES = 10000
N_EDGES = 320000
D = 128
D_EDGE = 16
D_GLOBAL = 16
N_GRAPHS = 8
DC = D + 16
NC, NS = 2, 16
NW = NC * NS
EPW = N_EDGES // NW
CH = 128
NROW = N_EDGES // CH
N_PAD = 10240
ROWS_PER_SUB = N_PAD // NS


def _mm_kernel(a_ref, b_ref, o_ref):
    o_ref[...] = jnp.dot(a_ref[...], b_ref[...], preferred_element_type=jnp.float32)


def _edge_pre_kernel(eat_ref, w_ref, b_ref, o_ref):
    o_ref[...] = lax.dot_general(
        eat_ref[...], w_ref[...], (((0,), (0,)), ((), ())),
        preferred_element_type=jnp.float32) + b_ref[...]


def _sc_body(xw1_hbm, eb_hbm, row_hbm, col_hbm, out_hbm,
             acc_sh, row_v, col_v, g_v, h_v, rs0, rs1, gsem, esem, ssem):
    cid = lax.axis_index("c")
    sid = lax.axis_index("s")
    wid = sid * NC + cid
    rsem = (rs0, rs1)

    zero16 = jnp.zeros((16,), jnp.float32)
    one16 = jnp.ones((16,), jnp.float32)

    @plsc.parallel_loop(0, CH, 1, unroll=4)
    def zfill(i):
        for j in range(DC // 16):
            h_v[i, pl.ds(j * 16, 16)] = zero16

    def zcopy(i, _):
        pltpu.sync_copy(h_v, acc_sh.at[pl.ds(sid * ROWS_PER_SUB + i * CH, CH), :])
        return 0
    lax.fori_loop(0, ROWS_PER_SUB // CH, zcopy, 0)

    @plsc.parallel_loop(0, CH, 1, unroll=4)
    def onefill(i):
        h_v[i, pl.ds(D, 16)] = one16
    plsc.subcore_barrier()

    nrows = jnp.where(wid < 4, NROW // NW + 1, NROW // NW)
    rstart = NROW // NW * wid + jnp.minimum(wid, 4)

    def idx_issue(r, b):
        pltpu.async_copy(row_hbm.at[pl.ds(r, 1), :], row_v.at[pl.ds(b, 1), :],
                         rsem[b])
        pltpu.async_copy(col_hbm.at[pl.ds(r, 1), :], col_v.at[pl.ds(b, 1), :],
                         rsem[b])

    idx_issue(rstart, 0)

    def chunk(c, _):
        r = rstart + c
        b = lax.rem(c, 2)
        pltpu.async_copy(eb_hbm.at[pl.ds(r * CH, CH), :],
                         h_v.at[:, pl.ds(0, D)], esem)
        for bb in range(2):
            @pl.when(b == bb)
            def _():
                pltpu.make_async_copy(row_hbm.at[pl.ds(0, 1), :],
                                      row_v.at[pl.ds(bb, 1), :], rsem[bb]).wait()
                pltpu.make_async_copy(col_hbm.at[pl.ds(0, 1), :],
                                      col_v.at[pl.ds(bb, 1), :], rsem[bb]).wait()
                pltpu.async_copy(xw1_hbm.at[row_v.at[bb]], g_v, gsem)

        @pl.when(c + 1 < nrows)
        def _():
            for bb in range(2):
                @pl.when(b == bb)
                def _():
                    idx_issue(r + 1, 1 - bb)

        pltpu.make_async_copy(eb_hbm.at[pl.ds(0, CH), :],
                              h_v.at[:, pl.ds(0, D)], esem).wait()
        pltpu.make_async_copy(xw1_hbm.at[row_v.at[0]], g_v, gsem).wait()

        @plsc.parallel_loop(0, CH, 1, unroll=4)
        def fuse(i):
            for j in range(D // 16):
                sl = pl.ds(j * 16, 16)
                h_v[i, sl] = jnp.maximum(h_v[i, sl] + g_v[i, sl], 0.0)

        for bb in range(2):
            @pl.when(b == bb)
            def _():
                pltpu.async_copy(h_v, acc_sh.at[col_v.at[bb]], ssem,
                                 add=True).wait()
        return 0
    lax.fori_loop(0, nrows, chunk, 0)

    plsc.subcore_barrier()
    pltpu.sync_copy(
        acc_sh.at[pl.ds(sid * ROWS_PER_SUB, ROWS_PER_SUB), :],
        out_hbm.at[cid, pl.ds(sid * ROWS_PER_SUB, ROWS_PER_SUB), :],
    )


def _node_mlp_kernel(x_ref, s_ref, batch_ref, u_ref, w2_ref, b2_ref,
                     w3_ref, b3_ref, w4_ref, b4_ref, o_ref):
    s = s_ref[0, :, :D] + s_ref[1, :, :D]
    c = (s_ref[0, :, D:D + 1] + s_ref[1, :, D:D + 1])
    inv = 1.0 / jnp.maximum(c, 1.0)
    mean = (jnp.dot(s, w2_ref[...], preferred_element_type=jnp.float32)
            + c * b2_ref[...]) * inv
    b = batch_ref[0, 0, :]
    oh = (b[:, None] == lax.broadcasted_iota(jnp.int32, (b.shape[0], N_GRAPHS), 1))
    uc = jnp.dot(u_ref[...], w3_ref[D + D:, :], preferred_element_type=jnp.float32)
    t = (jnp.dot(x_ref[...], w3_ref[:D, :], preferred_element_type=jnp.float32)
         + jnp.dot(mean, w3_ref[D:D + D, :], preferred_element_type=jnp.float32)
         + jnp.dot(oh.astype(jnp.float32), uc, preferred_element_type=jnp.float32)
         + b3_ref[...])
    o_ref[...] = (jnp.dot(jnp.maximum(t, 0.0), w4_ref[...],
                          preferred_element_type=jnp.float32) + b4_ref[...])


def kernel(x, edge_index, edge_attr, u, batch, W1, b1, W2, b2, W3, b3, W4, b4):
    row = edge_index[0].astype(jnp.int32)
    col = edge_index[1].astype(jnp.int32)
    W1a = W1[:D]
    W1b = W1[D:]

    xw1 = pl.pallas_call(
        _mm_kernel,
        grid=(5,),
        in_specs=[
            pl.BlockSpec((N_NODES // 5, D), lambda i: (i, 0)),
            pl.BlockSpec((D, D), lambda i: (0, 0)),
        ],
        out_specs=pl.BlockSpec((N_NODES // 5, D), lambda i: (i, 0)),
        out_shape=jax.ShapeDtypeStruct((N_NODES, D), jnp.float32),
    )(x, W1a)

    EB_BLK = 6400
    eb = pl.pallas_call(
        _edge_pre_kernel,
        grid=(N_EDGES // EB_BLK,),
        in_specs=[
            pl.BlockSpec((D_EDGE, EB_BLK), lambda i: (0, i)),
            pl.BlockSpec((D_EDGE, D), lambda i: (0, 0)),
            pl.BlockSpec((D,), lambda i: (0,)),
        ],
        out_specs=pl.BlockSpec((EB_BLK, D), lambda i: (i, 0)),
        out_shape=jax.ShapeDtypeStruct((N_EDGES, D), jnp.float32),
    )(edge_attr.T, W1b, b1)

    mesh = plsc.VectorSubcoreMesh(core_axis_name="c", subcore_axis_name="s")
    sc = functools.partial(
        pl.kernel,
        mesh=mesh,
        out_type=jax.ShapeDtypeStruct((NC, N_PAD, DC), jnp.float32),
        scratch_types=[
            pltpu.VMEM_SHARED((N_PAD, DC), jnp.float32),
            pltpu.VMEM((2, CH), jnp.int32),
            pltpu.VMEM((2, CH), jnp.int32),
            pltpu.VMEM((CH, D), jnp.float32),
            pltpu.VMEM((CH, DC), jnp.float32),
            pltpu.SemaphoreType.DMA,
            pltpu.SemaphoreType.DMA,
            pltpu.SemaphoreType.DMA,
            pltpu.SemaphoreType.DMA,
            pltpu.SemaphoreType.DMA,
        ],
        compiler_params=pltpu.CompilerParams(use_tc_tiling_on_sc=False),
    )(_sc_body)
    s01 = sc(xw1, eb, row.reshape(NROW, CH), col.reshape(NROW, CH))

    R = 1000
    batch3 = batch.astype(jnp.int32).reshape(N_NODES // R, 1, R)
    out = pl.pallas_call(
        _node_mlp_kernel,
        grid=(N_NODES // R,),
        in_specs=[
            pl.BlockSpec((R, D), lambda i: (i, 0)),
            pl.BlockSpec((NC, R, DC), lambda i: (0, i, 0)),
            pl.BlockSpec((1, 1, R), lambda i: (i, 0, 0)),
            pl.BlockSpec((N_GRAPHS, D_GLOBAL), lambda i: (0, 0)),
            pl.BlockSpec((D, D), lambda i: (0, 0)),
            pl.BlockSpec((D,), lambda i: (0,)),
            pl.BlockSpec((D + D + D_GLOBAL, D), lambda i: (0, 0)),
            pl.BlockSpec((D,), lambda i: (0,)),
            pl.BlockSpec((D, D), lambda i: (0, 0)),
            pl.BlockSpec((D,), lambda i: (0,)),
        ],
        out_specs=pl.BlockSpec((R, D), lambda i: (i, 0)),
        out_shape=jax.ShapeDtypeStruct((N_NODES, D), jnp.float32),
    )(x, s01, batch3, u, W2, b2, W3, b3, W4, b4)
    return out
